# Initial kernel scaffold; baseline (speedup 1.0000x reference)
import jax
import jax.numpy as jnp
from jax.experimental import pallas as pl


def kernel(x, embed_weight, pe):
    raise NotImplementedError("write your pallas kernel here")


# speedup vs baseline: 2.6130x; 2.6130x over previous
import jax
import jax.numpy as jnp
from jax.experimental import pallas as pl

_B, _P, _V, _C = 16384, 50, 39, 32
_ROWS = _B * _P
_BF = 12800
_NB = _ROWS // _BF


def _embed_body(x_ref, w_ref, pe_ref, o_ref):
    xb = x_ref[...]
    iota = jax.lax.broadcasted_iota(jnp.int32, (1, _V), 1)
    oh = (xb == iota).astype(jnp.float32)
    emb = jax.lax.dot_general(
        oh, w_ref[...], (((1,), (0,)), ((), ())),
        preferred_element_type=jnp.float32)
    o_ref[...] = emb + pe_ref[...]


def kernel(x, embed_weight, pe):
    xf = x.reshape(_ROWS, 1).astype(jnp.int32)
    pe_tiled = jnp.tile(pe[0], (_BF // _P, 1))
    out = pl.pallas_call(
        _embed_body,
        grid=(_NB,),
        in_specs=[
            pl.BlockSpec((_BF, 1), lambda i: (i, 0)),
            pl.BlockSpec((_V, _C), lambda i: (0, 0)),
            pl.BlockSpec((_BF, _C), lambda i: (0, 0)),
        ],
        out_specs=pl.BlockSpec((_BF, _C), lambda i: (i, 0)),
        out_shape=jax.ShapeDtypeStruct((_ROWS, _C), jnp.float32),
    )(xf, embed_weight, pe_tiled)
    return out.reshape(_B, _P, _C)


# speedup vs baseline: 3.3041x; 1.2645x over previous
import jax
import jax.numpy as jnp
import numpy as np
from jax import lax
from jax.experimental import pallas as pl
from jax.experimental.pallas import tpu as pltpu
from jax.experimental.pallas import tpu_sc as plsc

_B, _P, _V, _C = 16384, 50, 39, 32
_ROWS = _B * _P
_T = _P * _V
_W = 256
_NWIN = _ROWS // _W
_CH = 128
_NCH = _W // _CH
_OFFP = 25


def _table_body(w_ref, pe_ref, o_ref):
    o_ref[...] = w_ref[...] + pe_ref[...]


def _build_table(embed_weight, pe):
    w_flat = embed_weight.reshape(1, _V * _C)
    pe_tiled = jnp.tile(pe[0], (1, _V))
    t2 = pl.pallas_call(
        _table_body,
        out_shape=jax.ShapeDtypeStruct((_P, _V * _C), jnp.float32),
    )(w_flat, pe_tiled)
    return t2.reshape(_T, _C)


def _make_sc_kernel():
    mesh = plsc.VectorSubcoreMesh(core_axis_name="c", subcore_axis_name="s")

    def body(tbl_hbm, xi_hbm, off_hbm, out_hbm, tbl_sh, idx_v, sem):
        @pl.when(lax.axis_index("s") == 0)
        def _():
            pltpu.sync_copy(tbl_hbm, tbl_sh)

        plsc.subcore_barrier()

        def window(i_vmem, off_vmem, o_vmem):
            for j in range(_NCH):
                for k in range(0, _CH, 16):
                    col = j * _CH + k
                    idx_v[j, pl.ds(k, 16)] = (
                        i_vmem[0, pl.ds(col, 16)] + off_vmem[0, pl.ds(col, 16)]
                    )
            copies = [
                pltpu.async_copy(
                    tbl_sh.at[idx_v.at[j]],
                    o_vmem.at[pl.ds(j * _CH, _CH)],
                    sem,
                )
                for j in range(_NCH)
            ]
            for c in copies:
                c.wait()

        pltpu.emit_pipeline(
            window,
            grid=(_NWIN,),
            in_specs=[
                pl.BlockSpec((1, _W), lambda i: (0, i)),
                pl.BlockSpec((1, _W), lambda i: (lax.rem(i, _OFFP), 0)),
            ],
            out_specs=[pl.BlockSpec((_W, _C), lambda i: (i, 0))],
            core_axis_name=("c", "s"),
            dimension_semantics=(pltpu.PARALLEL,),
        )(xi_hbm, off_hbm, out_hbm)

    return pl.kernel(
        body,
        out_type=jax.ShapeDtypeStruct((_ROWS, _C), jnp.float32),
        mesh=mesh,
        scratch_types=[
            pltpu.VMEM_SHARED((_T, _C), jnp.float32),
            pltpu.VMEM((_NCH, _CH), jnp.int32),
            pltpu.SemaphoreType.DMA,
        ],
    )


def kernel(x, embed_weight, pe):
    tbl = _build_table(embed_weight, pe)
    xi = x.reshape(1, _ROWS).astype(jnp.int32)
    off = jnp.asarray(
        (_V * (np.arange(_OFFP * _W) % _P)).reshape(_OFFP, _W), dtype=jnp.int32
    )
    out = _make_sc_kernel()(tbl, xi, off)
    return out.reshape(_B, _P, _C)


# speedup vs baseline: 3.3341x; 1.0091x over previous
import jax
import jax.numpy as jnp
from jax import lax
from jax.experimental import pallas as pl
from jax.experimental.pallas import tpu as pltpu
from jax.experimental.pallas import tpu_sc as plsc

_B, _P, _V, _C = 16384, 50, 39, 32
_ROWS = _B * _P
_T = _P * _V
_NW = 32
_CH = 128
_S = 256
_SR = _S // _CH
_IR = _S // _C
_NST = _ROWS // (_NW * _S)


def _table_body(w_ref, pe_ref, o_ref):
    o_ref[...] = w_ref[...] + pe_ref[...]


def _build_table(embed_weight, pe):
    w_flat = embed_weight.reshape(1, _V * _C)
    pe_tiled = jnp.tile(pe[0], (1, _V))
    t2 = pl.pallas_call(
        _table_body,
        out_shape=jax.ShapeDtypeStruct((_P, _V * _C), jnp.float32),
    )(w_flat, pe_tiled)
    return t2.reshape(_T, _C)


def _make_sc_kernel():
    mesh = plsc.VectorSubcoreMesh(core_axis_name="c", subcore_axis_name="s")

    def body(tbl_hbm, xi_hbm, out_hbm,
             tbl_sh, ridx0, ridx1, idxf0, idxf1, stg0, stg1,
             semg, semwb0, semwb1):
        @pl.when(lax.axis_index("s") == 0)
        def _():
            pltpu.sync_copy(tbl_hbm, tbl_sh)

        plsc.subcore_barrier()

        wid = lax.axis_index("s") * 2 + lax.axis_index("c")
        iota16 = lax.iota(jnp.int32, 16)
        bufs = ((ridx0, idxf0, stg0, semwb0), (ridx1, idxf1, stg1, semwb1))

        def do_stage(st, b, not_first):
            not_first = jnp.asarray(not_first, dtype=jnp.bool_)
            ridx, idxf, stg, semwb = bufs[b]
            snum = wid * _NST + st
            irow0 = pl.multiple_of(snum * _IR, 8)
            row0 = pl.multiple_of(snum * _S, 8)
            pltpu.sync_copy(xi_hbm.at[pl.ds(irow0, _IR)], ridx)
            for n0 in range(0, _S, 16):
                rows = (row0 + n0) + iota16
                idxf[n0 // _CH, pl.ds(n0 % _CH, 16)] = (
                    ridx[n0 // _C, pl.ds(n0 % _C, 16)] + _V * lax.rem(rows, _P)
                )
            @pl.when(not_first)
            def _():
                pltpu.make_async_copy(
                    stg, out_hbm.at[pl.ds(0, _S)], semwb).wait()
            copies = [
                pltpu.async_copy(
                    tbl_sh.at[idxf.at[j]],
                    stg.at[pl.ds(j * _CH, _CH)],
                    semg,
                )
                for j in range(_SR)
            ]
            for c in copies:
                c.wait()
            pltpu.async_copy(stg, out_hbm.at[pl.ds(row0, _S)], semwb)

        @pl.loop(0, _NST // 2)
        def _(gg):
            do_stage(gg * 2, 0, gg > 0)
            do_stage(gg * 2 + 1, 1, gg > 0)

        pltpu.make_async_copy(stg0, out_hbm.at[pl.ds(0, _S)], semwb0).wait()
        pltpu.make_async_copy(stg1, out_hbm.at[pl.ds(0, _S)], semwb1).wait()

    return pl.kernel(
        body,
        out_type=jax.ShapeDtypeStruct((_ROWS, _C), jnp.float32),
        mesh=mesh,
        scratch_types=[
            pltpu.VMEM_SHARED((_T, _C), jnp.float32),
            pltpu.VMEM((_IR, _C), jnp.int32),
            pltpu.VMEM((_IR, _C), jnp.int32),
            pltpu.VMEM((_SR, _CH), jnp.int32),
            pltpu.VMEM((_SR, _CH), jnp.int32),
            pltpu.VMEM((_S, _C), jnp.float32),
            pltpu.VMEM((_S, _C), jnp.float32),
            pltpu.SemaphoreType.DMA,
            pltpu.SemaphoreType.DMA,
            pltpu.SemaphoreType.DMA,
        ],
    )


def kernel(x, embed_weight, pe):
    tbl = _build_table(embed_weight, pe)
    xi = x.reshape(_ROWS // _C, _C).astype(jnp.int32)
    out = _make_sc_kernel()(tbl, xi)
    return out.reshape(_B, _P, _C)


# speedup vs baseline: 52.0367x; 15.6075x over previous
import jax
import jax.numpy as jnp
from jax.experimental import pallas as pl

_B, _P, _V, _C = 16384, 50, 39, 32


def _table_body(w_ref, pe_ref, o_ref):
    o_ref[...] = w_ref[...] + pe_ref[...]


def _build_table_t(embed_weight, pe):
    w_flat = embed_weight.T.reshape(1, _C * _V)
    pe_rep = jnp.repeat(pe[0], _V, axis=1)
    tbl2 = pl.pallas_call(
        _table_body,
        out_shape=jax.ShapeDtypeStruct((_P, _C * _V), jnp.float32),
    )(w_flat, pe_rep)
    return tbl2.reshape(_P, _C, _V)


def _main_body(x_ref, t_ref, o_ref):
    xv = x_ref[0]
    iota_v = jax.lax.broadcasted_iota(jnp.int32, (_V, _B), 0)
    oht = (iota_v == xv).astype(jnp.float32)
    o_ref[0] = jax.lax.dot_general(
        t_ref[0], oht, (((1,), (0,)), ((), ())),
        preferred_element_type=jnp.float32)


def kernel(x, embed_weight, pe):
    tbl_t = _build_table_t(embed_weight, pe)
    xt = x.astype(jnp.int32).T.reshape(_P, 1, _B)
    out_t = pl.pallas_call(
        _main_body,
        grid=(_P,),
        in_specs=[
            pl.BlockSpec((1, 1, _B), lambda p: (p, 0, 0)),
            pl.BlockSpec((1, _C, _V), lambda p: (p, 0, 0)),
        ],
        out_specs=pl.BlockSpec((1, _C, _B), lambda p: (p, 0, 0)),
        out_shape=jax.ShapeDtypeStruct((_P, _C, _B), jnp.float32),
    )(xt, tbl_t)
    return out_t.transpose(2, 0, 1)
